# baseline (speedup 1.0000x reference)
import functools

import jax
import jax.numpy as jnp
import numpy as np
from jax import lax
from jax.experimental import pallas as pl
from jax.experimental.pallas import tpu as pltpu
from jax.experimental.pallas import tpu_sc as plsc

B = 8
C = 19
H = 512
W = 512
N = H * W
K = N // 4

BH = 128

NC = 2
NS = 16
NB = 4
TPB = 8
BPC = 2
SLICE = N // TPB
CHUNK = 8192
NCHUNK = SLICE // CHUNK
LANES = 16

BINS1 = 1024
SH1 = 22
BINS2 = 256
SH2 = 14
MASK2 = BINS2 - 1


def _ce_body(lg_ref, tg_ref, out_ref):
    x = lg_ref[0]
    t = tg_ref[0]
    m = jnp.max(x, axis=0)
    s = jnp.sum(jnp.exp(x - m[None]), axis=0)
    lse = m + jnp.log(s)
    cls = lax.broadcasted_iota(jnp.int32, x.shape, 0)
    xt = jnp.sum(jnp.where(cls == t[None], x, 0.0), axis=0)
    out_ref[0] = jnp.maximum(lse - xt, 0.0)


def _loss_map(logits, target, boff):
    return pl.pallas_call(
        _ce_body,
        grid=(NB, H // BH),
        in_specs=[
            pl.BlockSpec((1, C, BH, W), lambda b, h: (b + boff, 0, h, 0)),
            pl.BlockSpec((1, BH, W), lambda b, h: (b + boff, h, 0)),
        ],
        out_specs=pl.BlockSpec((1, BH, W), lambda b, h: (b, h, 0)),
        out_shape=jax.ShapeDtypeStruct((NB, H, W), jnp.float32),
    )(logits, target)


def _scan_desc(mc_ref, ms_ref, bins, kf):
    nch = bins // LANES
    lane = lax.broadcasted_iota(jnp.int32, (LANES,), 0)

    def body(j, carry):
        ccnt, csum, bstar, cabove, sabove = carry
        base = (nch - 1 - j) * LANES
        c = mc_ref[0, pl.ds(base, LANES)]
        s = ms_ref[0, pl.ds(base, LANES)]
        for p in range(1, TPB):
            c = c + mc_ref[p, pl.ds(base, LANES)]
            s = s + ms_ref[p, pl.ds(base, LANES)]
        crev = lax.rev(c, (0,))
        srev = lax.rev(s, (0,))
        cumc = plsc.cumsum(crev)
        cums = plsc.cumsum(srev)
        inc = ccnt + cumc
        above = inc - crev
        hit = (inc >= kf) & (above < kf)
        binvec = base + 15 - lane
        bstar = bstar + jnp.sum(jnp.where(hit, binvec, 0))
        cabove = cabove + jnp.sum(jnp.where(hit, above, 0.0))
        sabove = sabove + jnp.sum(jnp.where(hit, csum + cums - srev, 0.0))
        ccnt = ccnt + jnp.sum(c)
        csum = csum + jnp.sum(s)
        return ccnt, csum, bstar, cabove, sabove

    init = (jnp.float32(0), jnp.float32(0), jnp.int32(0),
            jnp.float32(0), jnp.float32(0))
    out = lax.fori_loop(0, nch, body, init)
    return out[2], out[3], out[4]


def _sc_topk_body(loss_hbm, out_hbm, chunk_a, chunk_b, cnt1, sum1, cnt2, sum2,
                  cbuf, mc1, ms1, mc2, ms2, obuf, vbuf, sem0, sem1,
                  sh_c1, sh_s1, sh_c2, sh_s2, sh_var):
    cid = lax.axis_index("c")
    sid = lax.axis_index("s")
    lb = sid // TPB
    part = sid % TPB
    batch = cid * BPC + lb
    lane = lax.broadcasted_iota(jnp.int32, (LANES,), 0)
    zeros16 = jnp.zeros((LANES,), jnp.float32)
    ones16 = jnp.ones((LANES,), jnp.float32)
    base_px = part * SLICE

    sems = (sem0, sem1)
    bufs = (chunk_a, chunk_b)

    def zero_loop(ref, n):
        def zb(i, _):
            ref[pl.ds(i * LANES, LANES)] = zeros16
            return 0
        lax.fori_loop(0, n // LANES, zb, 0, unroll=8)

    zero_loop(cnt1, 2 * BINS1 * LANES)
    zero_loop(sum1, 2 * BINS1 * LANES)
    zero_loop(cnt2, 2 * BINS2 * LANES)
    zero_loop(sum2, 2 * BINS2 * LANES)

    def chunk_src(ci):
        return loss_hbm.at[batch, pl.ds(base_px + ci * CHUNK, CHUNK)]

    def stream_chunks(process):
        pltpu.async_copy(chunk_src(0), bufs[0], sems[0])
        for ci in range(NCHUNK):
            if ci + 1 < NCHUNK:
                pltpu.async_copy(chunk_src(ci + 1), bufs[(ci + 1) % 2],
                                 sems[(ci + 1) % 2])
            pltpu.make_async_copy(chunk_src(ci), bufs[ci % 2],
                                  sems[ci % 2]).wait()
            process(bufs[ci % 2])

    def hist1(buf):
        rep = BINS1 * LANES

        def inner(i, _):
            for r in range(2):
                v = buf[pl.ds(i * 2 * LANES + r * LANES, LANES)]
                bits = plsc.bitcast(v, jnp.int32)
                b1 = lax.shift_right_logical(bits, SH1)
                idx = b1 * LANES + lane + r * rep
                plsc.addupdate_scatter(cnt1, [idx], ones16)
                plsc.addupdate_scatter(sum1, [idx], v)
            return 0

        lax.fori_loop(0, CHUNK // (2 * LANES), inner, 0, unroll=4)

    stream_chunks(hist1)

    def compact_and_stage(src_ref, dst_sh, bins):
        rep = bins * LANES

        def cb(i, _):
            acc = zeros16
            rowbase = i * LANES * LANES
            for l in range(LANES):
                pos = rowbase + lane * LANES + l
                acc = acc + plsc.load_gather(src_ref, [pos])
                acc = acc + plsc.load_gather(src_ref, [pos + rep])
            cbuf[pl.ds(i * LANES, LANES)] = acc
            return 0
        lax.fori_loop(0, bins // LANES, cb, 0)
        pltpu.sync_copy(cbuf.at[pl.ds(0, bins)], dst_sh.at[lb, part])

    def scan_and_publish(mc, ms, sh_c, sh_s, bins, kf, slot):
        pltpu.sync_copy(sh_c.at[lb], mc)
        pltpu.sync_copy(sh_s.at[lb], ms)
        return _scan_desc(mc, ms, bins, kf)

    compact_and_stage(cnt1, sh_c1, BINS1)
    compact_and_stage(sum1, sh_s1, BINS1)
    plsc.subcore_barrier()
    kf = jnp.float32(K)
    b1s, c1, s1 = scan_and_publish(mc1, ms1, sh_c1, sh_s1, BINS1, kf, 0)

    def hist2(buf):
        rep = BINS2 * LANES

        def inner(i, _):
            for r in range(2):
                v = buf[pl.ds(i * 2 * LANES + r * LANES, LANES)]
                bits = plsc.bitcast(v, jnp.int32)
                b1 = lax.shift_right_logical(bits, SH1)
                m = b1 == b1s
                digit = lax.shift_right_logical(bits, SH2) & MASK2
                idx = digit * LANES + lane + r * rep
                plsc.addupdate_scatter(cnt2, [idx], ones16, mask=m)
                plsc.addupdate_scatter(sum2, [idx], v, mask=m)
            return 0

        lax.fori_loop(0, CHUNK // (2 * LANES), inner, 0, unroll=4)

    stream_chunks(hist2)

    compact_and_stage(cnt2, sh_c2, BINS2)
    compact_and_stage(sum2, sh_s2, BINS2)
    plsc.subcore_barrier()
    b2s, c2, s2 = scan_and_publish(mc2, ms2, sh_c2, sh_s2, BINS2, kf - c1, 1)

    vf_bits = lax.shift_left(b1s, SH1) | lax.shift_left(b2s, SH2)
    vf = lax.bitcast_convert_type(vf_bits, jnp.float32)
    total = s1 + s2 + vf * (kf - c1 - c2)

    @pl.when(part == 0)
    def _():
        obuf[...] = jnp.broadcast_to(total, (LANES,))
        pltpu.sync_copy(obuf, out_hbm.at[batch])


def _sc_topk(loss_flat):
    mesh = plsc.VectorSubcoreMesh(core_axis_name="c", subcore_axis_name="s",
                                  num_cores=NC, num_subcores=NS)
    f32 = jnp.float32
    kern = pl.kernel(
        _sc_topk_body,
        out_type=jax.ShapeDtypeStruct((NB, LANES), f32),
        mesh=mesh,
        scratch_types=[
            pltpu.VMEM((CHUNK,), f32),
            pltpu.VMEM((CHUNK,), f32),
            pltpu.VMEM((2 * BINS1 * LANES,), f32),
            pltpu.VMEM((2 * BINS1 * LANES,), f32),
            pltpu.VMEM((2 * BINS2 * LANES,), f32),
            pltpu.VMEM((2 * BINS2 * LANES,), f32),
            pltpu.VMEM((BINS1,), f32),
            pltpu.VMEM((TPB, BINS1), f32),
            pltpu.VMEM((TPB, BINS1), f32),
            pltpu.VMEM((TPB, BINS2), f32),
            pltpu.VMEM((TPB, BINS2), f32),
            pltpu.VMEM((LANES,), f32),
            pltpu.VMEM((LANES,), f32),
            pltpu.SemaphoreType.DMA,
            pltpu.SemaphoreType.DMA,
            pltpu.VMEM_SHARED((BPC, TPB, BINS1), f32),
            pltpu.VMEM_SHARED((BPC, TPB, BINS1), f32),
            pltpu.VMEM_SHARED((BPC, TPB, BINS2), f32),
            pltpu.VMEM_SHARED((BPC, TPB, BINS2), f32),
            pltpu.VMEM_SHARED((BPC, 2, LANES), f32),
        ],
        compiler_params=pltpu.CompilerParams(needs_layout_passes=False),
    )
    return kern(loss_flat)


@jax.jit
def kernel(logits, target):
    tgt = target.astype(jnp.int32)
    loss0 = _loss_map(logits, tgt, 0)
    loss1 = _loss_map(logits, tgt, NB)
    o0 = _sc_topk(loss0.reshape(NB, N))
    o1 = _sc_topk(loss1.reshape(NB, N))
    return (jnp.sum(o0[:, 0]) + jnp.sum(o1[:, 0])) / jnp.float32(B * K)


# speedup vs baseline: 1.4846x; 1.4846x over previous
import functools

import jax
import jax.numpy as jnp
import numpy as np
from jax import lax
from jax.experimental import pallas as pl
from jax.experimental.pallas import tpu as pltpu
from jax.experimental.pallas import tpu_sc as plsc

B = 8
C = 19
H = 512
W = 512
N = H * W
K = N // 4

BH = 128

NC = 2
NS = 16
NB = 4
TPB = 8
BPC = 2
SLICE = N // TPB
CHUNK = 8192
NCHUNK = SLICE // CHUNK
LANES = 16

BINS1 = 1024
SH1 = 22
BINS2 = 256
SH2 = 14
MASK2 = BINS2 - 1


def _ce_body(lg_ref, tg_ref, out_ref):
    x = lg_ref[0]
    t = tg_ref[0]
    m = jnp.max(x, axis=0)
    s = jnp.sum(jnp.exp(x - m[None]), axis=0)
    lse = m + jnp.log(s)
    cls = lax.broadcasted_iota(jnp.int32, x.shape, 0)
    xt = jnp.sum(jnp.where(cls == t[None], x, 0.0), axis=0)
    out_ref[0] = jnp.maximum(lse - xt, 0.0)


def _loss_map(logits, target, boff):
    return pl.pallas_call(
        _ce_body,
        grid=(NB, H // BH),
        in_specs=[
            pl.BlockSpec((1, C, BH, W), lambda b, h: (b + boff, 0, h, 0)),
            pl.BlockSpec((1, BH, W), lambda b, h: (b + boff, h, 0)),
        ],
        out_specs=pl.BlockSpec((1, BH, W), lambda b, h: (b, h, 0)),
        out_shape=jax.ShapeDtypeStruct((NB, H, W), jnp.float32),
    )(logits, target)


def _scan_desc(mc_ref, ms_ref, bins, kf):
    nch = bins // LANES
    lane = lax.broadcasted_iota(jnp.int32, (LANES,), 0)

    def body(j, carry):
        ccnt, csum, bstar, cabove, sabove = carry
        base = (nch - 1 - j) * LANES
        c = mc_ref[0, pl.ds(base, LANES)]
        s = ms_ref[0, pl.ds(base, LANES)]
        for p in range(1, TPB):
            c = c + mc_ref[p, pl.ds(base, LANES)]
            s = s + ms_ref[p, pl.ds(base, LANES)]
        crev = lax.rev(c, (0,))
        srev = lax.rev(s, (0,))
        cumc = plsc.cumsum(crev)
        cums = plsc.cumsum(srev)
        inc = ccnt + cumc
        above = inc - crev
        hit = (inc >= kf) & (above < kf)
        binvec = base + 15 - lane
        bstar = bstar + jnp.sum(jnp.where(hit, binvec, 0))
        cabove = cabove + jnp.sum(jnp.where(hit, above, 0.0))
        sabove = sabove + jnp.sum(jnp.where(hit, csum + cums - srev, 0.0))
        ccnt = ccnt + jnp.sum(c)
        csum = csum + jnp.sum(s)
        return ccnt, csum, bstar, cabove, sabove

    init = (jnp.float32(0), jnp.float32(0), jnp.int32(0),
            jnp.float32(0), jnp.float32(0))
    out = lax.fori_loop(0, nch, body, init)
    return out[2], out[3], out[4]


def _sc_topk_body(loss_hbm, out_hbm, chunk_a, chunk_b, cnt1, sum1, cnt2, sum2,
                  cbuf, mc1, ms1, mc2, ms2, obuf, vbuf, sem0, sem1,
                  sh_c1, sh_s1, sh_c2, sh_s2, sh_var):
    cid = lax.axis_index("c")
    sid = lax.axis_index("s")
    lb = sid // TPB
    part = sid % TPB
    batch = cid * BPC + lb
    lane = lax.broadcasted_iota(jnp.int32, (LANES,), 0)
    zeros16 = jnp.zeros((LANES,), jnp.float32)
    ones16 = jnp.ones((LANES,), jnp.float32)
    base_px = part * SLICE

    sems = (sem0, sem1)
    bufs = (chunk_a, chunk_b)

    def zero_loop(ref, n):
        @plsc.parallel_loop(0, n // LANES, unroll=8)
        def _(i):
            ref[pl.ds(i * LANES, LANES)] = zeros16

    zero_loop(cnt1, BINS1 * LANES)
    zero_loop(sum1, BINS1 * LANES)
    zero_loop(cnt2, BINS2 * LANES)
    zero_loop(sum2, BINS2 * LANES)

    def chunk_src(ci):
        return loss_hbm.at[batch, pl.ds(base_px + ci * CHUNK, CHUNK)]

    def stream_chunks(process):
        pltpu.async_copy(chunk_src(0), bufs[0], sems[0])
        for ci in range(NCHUNK):
            if ci + 1 < NCHUNK:
                pltpu.async_copy(chunk_src(ci + 1), bufs[(ci + 1) % 2],
                                 sems[(ci + 1) % 2])
            pltpu.make_async_copy(chunk_src(ci), bufs[ci % 2],
                                  sems[ci % 2]).wait()
            process(bufs[ci % 2])

    def hist1(buf):
        @plsc.parallel_loop(0, CHUNK // LANES, unroll=8)
        def _(i):
            v = buf[pl.ds(i * LANES, LANES)]
            bits = plsc.bitcast(v, jnp.int32)
            b1 = lax.shift_right_logical(bits, SH1)
            idx = b1 * LANES + lane
            plsc.addupdate_scatter(cnt1, [idx], ones16)
            plsc.addupdate_scatter(sum1, [idx], v)

    stream_chunks(hist1)

    def compact_and_stage(src_ref, dst_sh, bins):
        @plsc.parallel_loop(0, bins // LANES, unroll=2)
        def _(i):
            acc = zeros16
            rowbase = i * LANES * LANES
            for l in range(LANES):
                acc = acc + plsc.load_gather(src_ref,
                                             [rowbase + lane * LANES + l])
            cbuf[pl.ds(i * LANES, LANES)] = acc
        pltpu.sync_copy(cbuf.at[pl.ds(0, bins)], dst_sh.at[lb, part])

    def scan_and_publish(mc, ms, sh_c, sh_s, bins, kf, slot):
        pltpu.sync_copy(sh_c.at[lb], mc)
        pltpu.sync_copy(sh_s.at[lb], ms)
        return _scan_desc(mc, ms, bins, kf)

    compact_and_stage(cnt1, sh_c1, BINS1)
    compact_and_stage(sum1, sh_s1, BINS1)
    plsc.subcore_barrier()
    kf = jnp.float32(K)
    b1s, c1, s1 = scan_and_publish(mc1, ms1, sh_c1, sh_s1, BINS1, kf, 0)

    def hist2(buf):
        @plsc.parallel_loop(0, CHUNK // LANES, unroll=8)
        def _(i):
            v = buf[pl.ds(i * LANES, LANES)]
            bits = plsc.bitcast(v, jnp.int32)
            b1 = lax.shift_right_logical(bits, SH1)
            m = b1 == b1s
            digit = lax.shift_right_logical(bits, SH2) & MASK2
            idx = digit * LANES + lane
            plsc.addupdate_scatter(cnt2, [idx], ones16, mask=m)
            plsc.addupdate_scatter(sum2, [idx], v, mask=m)

    stream_chunks(hist2)

    compact_and_stage(cnt2, sh_c2, BINS2)
    compact_and_stage(sum2, sh_s2, BINS2)
    plsc.subcore_barrier()
    b2s, c2, s2 = scan_and_publish(mc2, ms2, sh_c2, sh_s2, BINS2, kf - c1, 1)

    vf_bits = lax.shift_left(b1s, SH1) | lax.shift_left(b2s, SH2)
    vf = lax.bitcast_convert_type(vf_bits, jnp.float32)
    total = s1 + s2 + vf * (kf - c1 - c2)

    @pl.when(part == 0)
    def _():
        obuf[...] = jnp.broadcast_to(total, (LANES,))
        pltpu.sync_copy(obuf, out_hbm.at[batch])


def _sc_topk(loss_flat):
    mesh = plsc.VectorSubcoreMesh(core_axis_name="c", subcore_axis_name="s",
                                  num_cores=NC, num_subcores=NS)
    f32 = jnp.float32
    kern = pl.kernel(
        _sc_topk_body,
        out_type=jax.ShapeDtypeStruct((NB, LANES), f32),
        mesh=mesh,
        scratch_types=[
            pltpu.VMEM((CHUNK,), f32),
            pltpu.VMEM((CHUNK,), f32),
            pltpu.VMEM((BINS1 * LANES,), f32),
            pltpu.VMEM((BINS1 * LANES,), f32),
            pltpu.VMEM((BINS2 * LANES,), f32),
            pltpu.VMEM((BINS2 * LANES,), f32),
            pltpu.VMEM((BINS1,), f32),
            pltpu.VMEM((TPB, BINS1), f32),
            pltpu.VMEM((TPB, BINS1), f32),
            pltpu.VMEM((TPB, BINS2), f32),
            pltpu.VMEM((TPB, BINS2), f32),
            pltpu.VMEM((LANES,), f32),
            pltpu.VMEM((LANES,), f32),
            pltpu.SemaphoreType.DMA,
            pltpu.SemaphoreType.DMA,
            pltpu.VMEM_SHARED((BPC, TPB, BINS1), f32),
            pltpu.VMEM_SHARED((BPC, TPB, BINS1), f32),
            pltpu.VMEM_SHARED((BPC, TPB, BINS2), f32),
            pltpu.VMEM_SHARED((BPC, TPB, BINS2), f32),
            pltpu.VMEM_SHARED((BPC, 2, LANES), f32),
        ],
        compiler_params=pltpu.CompilerParams(needs_layout_passes=False),
    )
    return kern(loss_flat)


@jax.jit
def kernel(logits, target):
    tgt = target.astype(jnp.int32)
    loss0 = _loss_map(logits, tgt, 0)
    loss1 = _loss_map(logits, tgt, NB)
    o0 = _sc_topk(loss0.reshape(NB, N))
    o1 = _sc_topk(loss1.reshape(NB, N))
    return (jnp.sum(o0[:, 0]) + jnp.sum(o1[:, 0])) / jnp.float32(B * K)
